# baseline (speedup 1.0000x reference)
import numpy as np
import jax
import jax.numpy as jnp
from jax.experimental import pallas as pl
from jax.experimental.pallas import tpu as pltpu

_EPS = 1e-6
_P = 6.5
_BIN_NUM = (64,)


def _pow_p_static(z, p):
    two_p = round(2.0 * float(p))
    assert two_p > 0 and abs(2.0 * float(p) - two_p) < 1e-9
    n, has_half = two_p // 2, bool(two_p & 1)
    out, base, e = None, z, n
    while e > 0:
        if e & 1:
            out = base if out is None else out * base
        e >>= 1
        if e:
            base = base * base
    if has_half:
        r = jnp.sqrt(z)
        out = r if out is None else out * r
    return out


def _body(x_ref, a_ref, o_ref):
    z = jnp.maximum(x_ref[...].astype(jnp.float32), _EPS)
    zp = _pow_p_static(z, _P)
    m = jnp.dot(zp.astype(jnp.bfloat16), a_ref[...],
                preferred_element_type=jnp.float32)
    o_ref[...] = jnp.exp(jnp.log(m) * (1.0 / _P)).astype(o_ref.dtype)


def _averaging_matrix(hw, bin_num):
    blocks = []
    for b in bin_num:
        k = hw // b
        blocks.append(np.repeat(np.eye(b, dtype=np.float32), k, axis=0) / float(k))
    return jnp.asarray(np.concatenate(blocks, axis=1), dtype=jnp.bfloat16)


def kernel(x):
    n, c, h, w = x.shape
    hw = h * w
    for b in _BIN_NUM:
        assert hw % b == 0
    bins = sum(_BIN_NUM)
    rows = n * c

    x2 = x.reshape(rows, hw)
    avg_mat = _averaging_matrix(hw, _BIN_NUM)

    tr = 2048
    if rows % tr:
        tr = rows if rows < tr else next(
            t for t in (1024, 512, 256, 128, 64, 32, 16, 8, 1) if rows % t == 0)
    grid = (rows // tr,)

    out2 = pl.pallas_call(
        _body,
        out_shape=jax.ShapeDtypeStruct((rows, bins), x.dtype),
        grid=grid,
        in_specs=[
            pl.BlockSpec((tr, hw), lambda i: (i, 0)),
            pl.BlockSpec((hw, bins), lambda i: (0, 0)),
        ],
        out_specs=pl.BlockSpec((tr, bins), lambda i: (i, 0)),
        compiler_params=pltpu.CompilerParams(
            dimension_semantics=("parallel",),
            vmem_limit_bytes=48 * 1024 * 1024,
        ),
    )(x2, avg_mat)

    return out2.reshape(n, c, bins)


# speedup vs baseline: 4.0018x; 4.0018x over previous
import jax
import jax.numpy as jnp
from jax.experimental import pallas as pl
from jax.experimental.pallas import tpu as pltpu

_EPS = 1e-6
_P = 6.5
_BINS = 64


def _pow_p(z):
    z2 = z * z
    z4 = z2 * z2
    return (z2 * z4) * jnp.sqrt(z)


def _body(x_ref, o_ref):
    z = jnp.maximum(x_ref[...], _EPS)
    zp = _pow_p(z)
    bn, hw, ch = zp.shape
    zp4 = zp.reshape(bn, hw // 4, 4, ch)
    m = (zp4[:, :, 0, :] + zp4[:, :, 1, :] + zp4[:, :, 2, :] + zp4[:, :, 3, :]) * 0.25
    o_ref[...] = jnp.exp(jnp.log(m) * (1.0 / _P))


def kernel(x):
    n, c, h, w = x.shape
    hw = h * w
    assert hw % _BINS == 0 and hw // _BINS == 4

    x3 = x.transpose(0, 2, 3, 1).reshape(n, hw, c)

    bn = 4
    while n % bn:
        bn //= 2
    grid = (n // bn,)

    out3 = pl.pallas_call(
        _body,
        out_shape=jax.ShapeDtypeStruct((n, _BINS, c), x.dtype),
        grid=grid,
        in_specs=[pl.BlockSpec((bn, hw, c), lambda i: (i, 0, 0))],
        out_specs=pl.BlockSpec((bn, _BINS, c), lambda i: (i, 0, 0)),
        compiler_params=pltpu.CompilerParams(
            dimension_semantics=("parallel",),
            vmem_limit_bytes=48 * 1024 * 1024,
        ),
    )(x3)

    return out3.transpose(0, 2, 1)


# speedup vs baseline: 4.4749x; 1.1182x over previous
import jax
import jax.numpy as jnp
from jax.experimental import pallas as pl
from jax.experimental.pallas import tpu as pltpu

_EPS = 1e-6
_P = 6.5
_BINS = 64


def _body(x_ref, o_ref):
    z = jnp.maximum(x_ref[...], _EPS)
    zp = jnp.exp2(_P * jnp.log2(z))
    bn, hw, ch = zp.shape
    zp4 = zp.reshape(bn, hw // 4, 4, ch)
    m = zp4[:, :, 0, :] + zp4[:, :, 1, :] + zp4[:, :, 2, :] + zp4[:, :, 3, :]
    o_ref[...] = jnp.exp2((jnp.log2(m) - 2.0) * (1.0 / _P))


def kernel(x):
    n, c, h, w = x.shape
    hw = h * w
    assert hw % _BINS == 0 and hw // _BINS == 4

    x3 = x.transpose(0, 2, 3, 1).reshape(n, hw, c)

    bn = 4
    while n % bn:
        bn //= 2
    grid = (n // bn,)

    out3 = pl.pallas_call(
        _body,
        out_shape=jax.ShapeDtypeStruct((n, _BINS, c), x.dtype),
        grid=grid,
        in_specs=[pl.BlockSpec((bn, hw, c), lambda i: (i, 0, 0))],
        out_specs=pl.BlockSpec((bn, _BINS, c), lambda i: (i, 0, 0)),
        compiler_params=pltpu.CompilerParams(
            dimension_semantics=("parallel",),
            vmem_limit_bytes=48 * 1024 * 1024,
        ),
    )(x3)

    return out3.transpose(0, 2, 1)


# speedup vs baseline: 5.8086x; 1.2980x over previous
import numpy as np
import jax
import jax.numpy as jnp
from jax.experimental import pallas as pl
from jax.experimental.pallas import tpu as pltpu

_EPS = 1e-6
_P = 6.5
_BINS = 64


def _body(a_ref, x_ref, o_ref):
    z = jnp.maximum(x_ref[...], _EPS)
    zp = jnp.exp2(_P * jnp.log2(z)).astype(jnp.bfloat16)
    a = a_ref[...]
    for b in range(x_ref.shape[0]):
        m = jax.lax.dot_general(a, zp[b], (((1,), (0,)), ((), ())),
                                preferred_element_type=jnp.float32)
        o_ref[b, :, :] = jnp.exp2(jnp.log2(m) * (1.0 / _P))


def _pool_matrix(hw):
    k = hw // _BINS
    a = np.repeat(np.eye(_BINS, dtype=np.float32), k, axis=0).T / float(k)
    return jnp.asarray(a, dtype=jnp.bfloat16)


def kernel(x):
    n, c, h, w = x.shape
    hw = h * w
    assert hw % _BINS == 0

    x3 = x.transpose(0, 2, 3, 1).reshape(n, hw, c)

    bn = 4
    while n % bn:
        bn //= 2
    grid = (n // bn,)

    out3 = pl.pallas_call(
        _body,
        out_shape=jax.ShapeDtypeStruct((n, _BINS, c), x.dtype),
        grid=grid,
        in_specs=[
            pl.BlockSpec((_BINS, hw), lambda i: (0, 0)),
            pl.BlockSpec((bn, hw, c), lambda i: (i, 0, 0)),
        ],
        out_specs=pl.BlockSpec((bn, _BINS, c), lambda i: (i, 0, 0)),
        compiler_params=pltpu.CompilerParams(
            dimension_semantics=("parallel",),
            vmem_limit_bytes=48 * 1024 * 1024,
        ),
    )(_pool_matrix(hw), x3)

    return out3.transpose(0, 2, 1)


# speedup vs baseline: 6.9817x; 1.2020x over previous
import numpy as np
import jax
import jax.numpy as jnp
from jax.experimental import pallas as pl
from jax.experimental.pallas import tpu as pltpu

_EPS = 1e-6
_P = 6.5
_BINS = 64


def _body(a_ref, x_ref, o_ref):
    z = jnp.maximum(x_ref[...], _EPS)
    zp = jnp.exp2(_P * jnp.log2(z)).astype(jnp.bfloat16)
    a = a_ref[...]
    for b in range(x_ref.shape[0]):
        m = jax.lax.dot_general(a, zp[b], (((1,), (0,)), ((), ())),
                                preferred_element_type=jnp.float32)
        o_ref[b, :, :] = jnp.exp2(jnp.log2(m) * (1.0 / _P))


def _pool_matrix(hw):
    k = hw // _BINS
    a = np.repeat(np.eye(_BINS, dtype=np.float32), k, axis=0).T / float(k)
    return jnp.asarray(a, dtype=jnp.bfloat16)


def kernel(x):
    n, c, h, w = x.shape
    hw = h * w
    assert hw % _BINS == 0

    x3 = x.transpose(0, 2, 3, 1).reshape(n, hw, c)

    bn = 8
    while n % bn:
        bn //= 2
    grid = (n // bn,)

    out3 = pl.pallas_call(
        _body,
        out_shape=jax.ShapeDtypeStruct((n, _BINS, c), x.dtype),
        grid=grid,
        in_specs=[
            pl.BlockSpec((_BINS, hw), lambda i: (0, 0)),
            pl.BlockSpec((bn, hw, c), lambda i: (i, 0, 0)),
        ],
        out_specs=pl.BlockSpec((bn, _BINS, c), lambda i: (i, 0, 0)),
        compiler_params=pltpu.CompilerParams(
            dimension_semantics=("parallel",),
            vmem_limit_bytes=48 * 1024 * 1024,
        ),
    )(_pool_matrix(hw), x3)

    return out3.transpose(0, 2, 1)


# speedup vs baseline: 7.3869x; 1.0580x over previous
import numpy as np
import jax
import jax.numpy as jnp
from jax.experimental import pallas as pl
from jax.experimental.pallas import tpu as pltpu

_EPS = 1e-6
_P = 6.5
_BINS = 64


def _body(a_ref, x_ref, o_ref):
    z = jnp.maximum(x_ref[...], _EPS)
    zp = jnp.exp2(_P * jnp.log2(z)).astype(jnp.bfloat16)
    a = a_ref[...]
    for b in range(x_ref.shape[0]):
        m = jax.lax.dot_general(a, zp[b], (((1,), (0,)), ((), ())),
                                preferred_element_type=jnp.float32)
        o_ref[b, :, :] = jnp.exp2(jnp.log2(m) * (1.0 / _P))


def _pool_matrix(hw):
    k = hw // _BINS
    a = np.repeat(np.eye(_BINS, dtype=np.float32), k, axis=0).T / float(k)
    return jnp.asarray(a, dtype=jnp.bfloat16)


def kernel(x):
    n, c, h, w = x.shape
    hw = h * w
    assert hw % _BINS == 0

    x3 = x.transpose(0, 2, 3, 1).reshape(n, hw, c)

    bn = 16
    while n % bn:
        bn //= 2
    grid = (n // bn,)

    out3 = pl.pallas_call(
        _body,
        out_shape=jax.ShapeDtypeStruct((n, _BINS, c), x.dtype),
        grid=grid,
        in_specs=[
            pl.BlockSpec((_BINS, hw), lambda i: (0, 0)),
            pl.BlockSpec((bn, hw, c), lambda i: (i, 0, 0)),
        ],
        out_specs=pl.BlockSpec((bn, _BINS, c), lambda i: (i, 0, 0)),
        compiler_params=pltpu.CompilerParams(
            dimension_semantics=("parallel",),
            vmem_limit_bytes=48 * 1024 * 1024,
        ),
    )(_pool_matrix(hw), x3)

    return out3.transpose(0, 2, 1)
